# baseline (speedup 1.0000x reference)
import jax
import jax.numpy as jnp
import numpy as np
from jax import lax
from jax.experimental import pallas as pl

B, N, D_ROI, SUB, H, NLP = 4, 100, 4096, 256, 64, 64
PRE_NMS = 6000
NMS_THRESH = 0.7


def _passthrough(a, b, c):
    def body(a_ref, b_ref, c_ref, oa_ref, ob_ref, oc_ref):
        oa_ref[...] = a_ref[...]
        ob_ref[...] = b_ref[...]
        oc_ref[...] = c_ref[...]
    Bv, Pv = a.shape[0], a.shape[1]
    return pl.pallas_call(
        body,
        grid=(Bv,),
        in_specs=[
            pl.BlockSpec((1, Pv, a.shape[2]), lambda i: (i, 0, 0)),
            pl.BlockSpec((1, Pv, b.shape[2]), lambda i: (i, 0, 0)),
            pl.BlockSpec((1, Pv, c.shape[2]), lambda i: (i, 0, 0)),
        ],
        out_specs=(
            pl.BlockSpec((1, Pv, a.shape[2]), lambda i: (i, 0, 0)),
            pl.BlockSpec((1, Pv, b.shape[2]), lambda i: (i, 0, 0)),
            pl.BlockSpec((1, Pv, c.shape[2]), lambda i: (i, 0, 0)),
        ),
        out_shape=(
            jax.ShapeDtypeStruct(a.shape, a.dtype),
            jax.ShapeDtypeStruct(b.shape, b.dtype),
            jax.ShapeDtypeStruct(c.shape, c.dtype),
        ),
    )(a, b, c)


def kernel(rois, im_info, roi_feat, nlp_features, W_sub, b_sub, W_ih_f, W_hh_f, b_ih_f, b_hh_f, W_ih_b, W_hh_b, b_ih_b, b_hh_b, W_lo, b_lo):
    Bsz, Nroi = rois.shape[0], rois.shape[1]
    ii = np.repeat(np.arange(Nroi), Nroi)
    jj = np.tile(np.arange(Nroi), Nroi)
    m = ii != jj
    pairs = jnp.asarray(np.stack([ii[m], jj[m]], axis=1))
    P = int(pairs.shape[0])
    K = min(PRE_NMS, P)
    feat = jnp.einsum('bnd,sd->bns', roi_feat, W_sub) + b_sub

    def cell(x, h, c, Wi, Wh, bi, bh):
        g = x @ Wi.T + h @ Wh.T + bi + bh
        i = jax.nn.sigmoid(g[:, :H])
        f = jax.nn.sigmoid(g[:, H:2 * H])
        gg = jnp.tanh(g[:, 2 * H:3 * H])
        o = jax.nn.sigmoid(g[:, 3 * H:])
        c2 = f * c + i * gg
        return o * jnp.tanh(c2), c2

    def iou_row(b, boxes, areas, area_b):
        xx1 = jnp.maximum(b[0], boxes[:, 0])
        yy1 = jnp.maximum(b[1], boxes[:, 1])
        xx2 = jnp.minimum(b[2], boxes[:, 2])
        yy2 = jnp.minimum(b[3], boxes[:, 3])
        inter = jnp.maximum(xx2 - xx1 + 1.0, 0.0) * jnp.maximum(yy2 - yy1 + 1.0, 0.0)
        return inter / (area_b + areas - inter)

    def greedy_nms(sb, ob):
        a_s = (sb[:, 2] - sb[:, 0] + 1.0) * (sb[:, 3] - sb[:, 1] + 1.0)
        a_o = (ob[:, 2] - ob[:, 0] + 1.0) * (ob[:, 3] - ob[:, 1] + 1.0)
        idx = jnp.arange(K)
        def body(i, keep):
            iou = jnp.minimum(iou_row(sb[i], sb, a_s, a_s[i]), iou_row(ob[i], ob, a_o, a_o[i]))
            sup = (iou > NMS_THRESH) & (idx > i) & keep[i]
            return keep & (~sup)
        return lax.fori_loop(0, K, body, jnp.ones((K,), bool))

    def per_batch(feat_b, rois_b, nlp_b, b_id):
        subj = feat_b[pairs[:, 0]]
        obj = feat_b[pairs[:, 1]]
        h0 = jnp.zeros((P, H), jnp.float32)
        c0 = jnp.zeros((P, H), jnp.float32)
        hf, cf = cell(subj, h0, c0, W_ih_f, W_hh_f, b_ih_f, b_hh_f)
        hf, cf = cell(obj, hf, cf, W_ih_f, W_hh_f, b_ih_f, b_hh_f)
        hb, _ = cell(obj, h0, c0, W_ih_b, W_hh_b, b_ih_b, b_hh_b)
        out_last = jnp.concatenate([hf, hb], axis=1)
        sel = out_last @ W_lo.T + b_lo
        num = (sel * nlp_b[None, :]).sum(1)
        den = jnp.maximum(jnp.linalg.norm(sel, axis=1) * jnp.linalg.norm(nlp_b), 1e-6)
        scores = num / den
        order = jnp.argsort(-scores)[:K]
        prop_s = pairs[order]
        sc_s = scores[order]
        sb = rois_b[prop_s[:, 0], 1:5]
        ob = rois_b[prop_s[:, 1], 1:5]
        keep = greedy_nms(lax.stop_gradient(sb), lax.stop_gradient(ob))
        pack = jnp.argsort(~keep)
        valid = jnp.arange(K) < keep.sum()
        prop_k = prop_s[pack]
        sc_k = sc_s[pack]
        sb_k = rois_b[prop_k[:, 0], 1:5] * valid[:, None]
        ob_k = rois_b[prop_k[:, 1], 1:5] * valid[:, None]
        col0 = jnp.where(valid, b_id.astype(jnp.float32), 0.0)[:, None]
        rows = jnp.concatenate([col0, sb_k, ob_k], axis=1)
        out = jnp.zeros((P, 9), jnp.float32).at[:K].set(rows)
        out_sc = jnp.zeros((P, 1), jnp.float32).at[:K].set(jnp.where(valid, sc_k, 0.0)[:, None])
        out_pr = jnp.zeros((P, 2), jnp.int32).at[:K].set(jnp.where(valid[:, None], prop_k, 0).astype(jnp.int32))
        return out, out_pr, out_sc

    out, out_pr, out_sc = jax.vmap(per_batch)(feat, rois, nlp_features, jnp.arange(Bsz))
    return _passthrough(out, out_pr, out_sc)


# speedup vs baseline: 70.5245x; 70.5245x over previous
import functools

import jax
import jax.numpy as jnp
import numpy as np
from jax import lax
from jax.experimental import pallas as pl
from jax.experimental.pallas import tpu as pltpu
from jax.experimental.pallas import tpu_sc as plsc

B, N, D_ROI, SUB, H, NLP = 4, 100, 4096, 256, 64, 64
PRE_NMS = 6000
NMS_THRESH = 0.7
NEG = -3.0e38
NPAD = 104
CI = 8


def _tree64(p):
    acc = p[:, 0:8]
    for k in range(1, 8):
        acc = acc + p[:, 8 * k:8 * k + 8]
    t = acc[:, 0:4] + acc[:, 4:8]
    t = t[:, 0:2] + t[:, 2:4]
    return t[:, 0:1] + t[:, 1:2]


def _gates(g):
    i = jax.nn.sigmoid(g[:, :H])
    f = jax.nn.sigmoid(g[:, H:2 * H])
    gg = jnp.tanh(g[:, 2 * H:3 * H])
    o = jax.nn.sigmoid(g[:, 3 * H:])
    return i, f, gg, o


def _feat_call(roi_feat, W_sub, b_sub):
    M = B * N
    def body(rf, ws, bs, o):
        o[...] = jnp.dot(rf[...], ws[...].T, preferred_element_type=jnp.float32) + bs[...]
    return pl.pallas_call(
        body, out_shape=jax.ShapeDtypeStruct((M, SUB), jnp.float32),
    )(roi_feat.reshape(M, D_ROI), W_sub, b_sub.reshape(1, SUB))


def _cell1_call(x, Wi, bi, bh):
    M = x.shape[0]
    def body(x_r, wi_r, bi_r, bh_r, ax_o, c_o, h_o):
        ax = jnp.dot(x_r[...], wi_r[...].T, preferred_element_type=jnp.float32)
        ax_o[...] = ax
        g1 = ax + bi_r[...] + bh_r[...]
        i1, f1, gg1, o1 = _gates(g1)
        c1 = i1 * gg1
        c_o[...] = c1
        h_o[...] = o1 * jnp.tanh(c1)
    return pl.pallas_call(
        body,
        out_shape=(
            jax.ShapeDtypeStruct((M, 4 * H), jnp.float32),
            jax.ShapeDtypeStruct((M, H), jnp.float32),
            jax.ShapeDtypeStruct((M, H), jnp.float32),
        ),
    )(x, Wi, bi.reshape(1, 4 * H), bh.reshape(1, 4 * H))


def _hmat_call(h, Wh):
    M = h.shape[0]
    def body(h_r, wh_r, o):
        o[...] = jnp.dot(h_r[...], wh_r[...].T, preferred_element_type=jnp.float32)
    return pl.pallas_call(
        body, out_shape=jax.ShapeDtypeStruct((M, 4 * H), jnp.float32),
    )(h, Wh)


def _nn_call(nlp):
    def body(nl, o):
        q = nl[...] * nl[...]
        o[...] = jnp.sqrt(_tree64(q))
    return pl.pallas_call(
        body, out_shape=jax.ShapeDtypeStruct((B, 1), jnp.float32),
    )(nlp)


def _pairwise_call(ax, bh_p, cf_p, hb, nlp3, nn3, W_lo, b_lo, b_ih_f, b_hh_f):
    def body(ax_r, bh_r, cf_r, hb_r, nl_r, nn_r, wlo_r, blo_r, bif_r, bhf_r, sc_o):
        ib = pl.program_id(1)
        axb = ax_r[0]
        bhc = bh_r[0]
        g2 = axb[None, :, :] + bhc[:, None, :]
        g2 = g2.reshape(CI * N, 4 * H) + bif_r[...] + bhf_r[...]
        i2, f2, gg2, o2 = _gates(g2)
        cfb = jnp.broadcast_to(cf_r[0][:, None, :], (CI, N, H)).reshape(CI * N, H)
        c2 = f2 * cfb + i2 * gg2
        h2 = o2 * jnp.tanh(c2)
        hbb = jnp.broadcast_to(hb_r[0][None, :, :], (CI, N, H)).reshape(CI * N, H)
        out_last = jnp.concatenate([h2, hbb], axis=1)
        sel = jnp.dot(out_last, wlo_r[...].T, preferred_element_type=jnp.float32) + blo_r[...]
        num = _tree64(sel * nl_r[0])
        ns = jnp.sqrt(_tree64(sel * sel))
        den = jnp.maximum(ns * nn_r[0], 1e-6)
        sc = (num / den).reshape(CI, N)
        row = jax.lax.broadcasted_iota(jnp.int32, (CI, N), 0) + ib * CI
        col = jax.lax.broadcasted_iota(jnp.int32, (CI, N), 1)
        sc = jnp.where(row == col, NEG, sc)
        sc_o[...] = sc[None]

    return pl.pallas_call(
        body,
        grid=(B, NPAD // CI),
        in_specs=[
            pl.BlockSpec((1, N, SUB), lambda b, i: (b, 0, 0)),
            pl.BlockSpec((1, CI, SUB), lambda b, i: (b, i, 0)),
            pl.BlockSpec((1, CI, H), lambda b, i: (b, i, 0)),
            pl.BlockSpec((1, N, H), lambda b, i: (b, 0, 0)),
            pl.BlockSpec((1, 1, NLP), lambda b, i: (b, 0, 0)),
            pl.BlockSpec((1, 1, 1), lambda b, i: (b, 0, 0)),
            pl.BlockSpec((H, 2 * H), lambda b, i: (0, 0)),
            pl.BlockSpec((1, H), lambda b, i: (0, 0)),
            pl.BlockSpec((1, 4 * H), lambda b, i: (0, 0)),
            pl.BlockSpec((1, 4 * H), lambda b, i: (0, 0)),
        ],
        out_specs=pl.BlockSpec((1, CI, N), lambda b, i: (b, i, 0)),
        out_shape=jax.ShapeDtypeStruct((B, NPAD, N), jnp.float32),
    )(ax, bh_p, cf_p, hb, nlp3, nn3, W_lo, b_lo.reshape(1, H),
      b_ih_f.reshape(1, 4 * H), b_hh_f.reshape(1, 4 * H))


def _scores_pallas(roi_feat, nlp_features, W_sub, b_sub, W_ih_f, W_hh_f, b_ih_f, b_hh_f,
                   W_ih_b, W_hh_b, b_ih_b, b_hh_b, W_lo, b_lo):
    feat = _feat_call(roi_feat, W_sub, b_sub)
    ax, cf1, h1 = _cell1_call(feat, W_ih_f, b_ih_f, b_hh_f)
    bh = _hmat_call(h1, W_hh_f)
    _, _, hb1 = _cell1_call(feat, W_ih_b, b_ih_b, b_hh_b)
    nn = _nn_call(nlp_features)
    ax = ax.reshape(B, N, SUB)
    bh_p = jnp.pad(bh.reshape(B, N, SUB), ((0, 0), (0, NPAD - N), (0, 0)))
    cf_p = jnp.pad(cf1.reshape(B, N, H), ((0, 0), (0, NPAD - N), (0, 0)))
    hb = hb1.reshape(B, N, H)
    grid = _pairwise_call(ax, bh_p, cf_p, hb, nlp_features.reshape(B, 1, NLP),
                          nn.reshape(B, 1, 1), W_lo, b_lo, b_ih_f, b_hh_f)
    return grid[:, :N, :].reshape(B, N * N)


RPAD = 10240
KP = 6016
NBLK = KP // 128
DUMP = 100000


def _rank_call(s_col, s_row):
    TB = 1024
    def body(sc_r, sr_r, o_r):
        blk = pl.program_id(1)
        tgt = sc_r[0]
        srow = sr_r[0]
        tgt_idx = blk * TB + jax.lax.broadcasted_iota(jnp.int32, (TB, 1), 0)
        acc = jnp.zeros((TB, 128), jnp.float32)
        for r in range(RPAD // 128):
            sl = srow[:, r * 128:(r + 1) * 128]
            src_idx = r * 128 + jax.lax.broadcasted_iota(jnp.int32, (1, 128), 1)
            gt = sl > tgt
            tie = (sl == tgt) & (src_idx < tgt_idx)
            acc = acc + jnp.where(gt | tie, 1.0, 0.0)
        rk = jnp.sum(acc, axis=1, keepdims=True)
        o_r[...] = rk.astype(jnp.int32)[None]

    return pl.pallas_call(
        body,
        grid=(B, RPAD // TB),
        in_specs=[
            pl.BlockSpec((1, TB, 1), lambda b, i: (b, i, 0)),
            pl.BlockSpec((1, 1, RPAD), lambda b, i: (b, 0, 0)),
        ],
        out_specs=pl.BlockSpec((1, TB, 1), lambda b, i: (b, i, 0)),
        out_shape=jax.ShapeDtypeStruct((B, RPAD, 1), jnp.int32),
    )(s_col, s_row)


def _sc_rank_scatter(rank, scores):
    mesh = plsc.VectorSubcoreMesh(core_axis_name="c", subcore_axis_name="s")

    @functools.partial(
        pl.kernel, mesh=mesh,
        compiler_params=pltpu.CompilerParams(needs_layout_passes=False),
        out_type=(jax.ShapeDtypeStruct((B, RPAD), jnp.int32),
                  jax.ShapeDtypeStruct((B, RPAD), jnp.float32)),
        scratch_types=[pltpu.VMEM((RPAD,), jnp.int32),
                       pltpu.VMEM((RPAD,), jnp.float32),
                       pltpu.VMEM((RPAD,), jnp.int32),
                       pltpu.VMEM((RPAD,), jnp.float32)],
    )
    def k(rank_hbm, sc_hbm, og_hbm, os_hbm, rk_v, sc_v, og_v, os_v):
        wid = lax.axis_index("s") * 2 + lax.axis_index("c")

        @pl.when(wid < B)
        def _():
            pltpu.sync_copy(rank_hbm.at[wid], rk_v)
            pltpu.sync_copy(sc_hbm.at[wid], sc_v)

            def body(c, carry):
                idx = rk_v[pl.ds(c * 16, 16)]
                vals = lax.iota(jnp.int32, 16) + c * 16
                plsc.store_scatter(og_v, [idx], vals)
                sv = sc_v[pl.ds(c * 16, 16)]
                plsc.store_scatter(os_v, [idx], sv)
                return carry

            lax.fori_loop(0, RPAD // 16, body, 0)
            pltpu.sync_copy(og_v, og_hbm.at[wid])
            pltpu.sync_copy(os_v, os_hbm.at[wid])

    return k(rank, scores)


def _nms_call(srt_mat, srt_matT, rois, roisT):
    def body(sm_r, smT_r, rois_r, roisT_r, pt_o, nk_o):
        r5 = rois_r[0]
        r5T = roisT_r[0]
        x1c, y1c, x2c, y2c = r5[:, 1:2], r5[:, 2:3], r5[:, 3:4], r5[:, 4:5]
        x1r, y1r, x2r, y2r = r5T[1:2, :], r5T[2:3, :], r5T[3:4, :], r5T[4:5, :]
        area_c = (x2c - x1c + 1.0) * (y2c - y1c + 1.0)
        area_r = (x2r - x1r + 1.0) * (y2r - y1r + 1.0)
        xx1 = jnp.maximum(x1c, x1r)
        yy1 = jnp.maximum(y1c, y1r)
        xx2 = jnp.minimum(x2c, x2r)
        yy2 = jnp.minimum(y2c, y2r)
        inter = jnp.maximum(xx2 - xx1 + 1.0, 0.0) * jnp.maximum(yy2 - yy1 + 1.0, 0.0)
        iou = inter / (area_c + area_r - inter)
        A = jnp.where(iou > NMS_THRESH, 1.0, 0.0)

        g_rows = sm_r[0]
        g_cols = smT_r[0]
        iota_rowN = jax.lax.broadcasted_iota(jnp.int32, (1, N), 1)
        iota_colN = jax.lax.broadcasted_iota(jnp.int32, (N, 1), 0)
        lane128 = jax.lax.broadcasted_iota(jnp.int32, (1, 128), 1)
        sub128 = jax.lax.broadcasted_iota(jnp.int32, (128, 1), 0)
        low_strict = jnp.where(sub128 > lane128, 1.0, 0.0)

        Supp = jnp.zeros((N, N), jnp.float32)
        keeps = []
        for t in range(NBLK):
            gi_col = g_cols[:, t:t + 1] // N
            gj_col = g_cols[:, t:t + 1] % N
            OiT = jnp.where(iota_colN == (g_rows[t:t + 1, :] // N), 1.0, 0.0)
            OjT = jnp.where(iota_colN == (g_rows[t:t + 1, :] % N), 1.0, 0.0)
            Oi = jnp.where(gi_col == iota_rowN, 1.0, 0.0)
            Oj = jnp.where(gj_col == iota_rowN, 1.0, 0.0)
            OiA = jnp.dot(Oi, A, preferred_element_type=jnp.float32)
            OjA = jnp.dot(Oj, A, preferred_element_type=jnp.float32)
            S = (jnp.dot(OiA, OiT, preferred_element_type=jnp.float32)
                 * jnp.dot(OjA, OjT, preferred_element_type=jnp.float32))
            su = jnp.sum(OiT * jnp.dot(Supp, OjT, preferred_element_type=jnp.float32),
                         axis=0, keepdims=True)
            pad = jnp.where(t * 128 + lane128 >= PRE_NMS, 1.0, 0.0)
            su = su + pad

            conflicts = jnp.sum(S * low_strict)

            def fast(su_, S_):
                return jnp.where(su_ == 0.0, 1.0, 0.0)

            def slow(su_, S_):
                def step(c, carry):
                    supvec, keepvec = carry
                    e_c = jnp.where(lane128 == c, 1.0, 0.0)
                    sup_c = jnp.sum(supvec * e_c)
                    k_c = jnp.where(sup_c == 0.0, 1.0, 0.0)
                    s_row = jnp.dot(e_c, S_, preferred_element_type=jnp.float32)
                    supvec = supvec + k_c * s_row
                    keepvec = keepvec + k_c * e_c
                    return supvec, keepvec
                _, kv = lax.fori_loop(0, 128, step, (su_, jnp.zeros((1, 128), jnp.float32)))
                return kv

            keep_row = lax.cond(conflicts == 0.0, fast, slow, su, S)
            keeps.append(keep_row)
            UiT = jnp.dot(A, OiT, preferred_element_type=jnp.float32) * keep_row
            Supp = Supp + jnp.dot(UiT, OjA, preferred_element_type=jnp.float32)

        keep_mat = jnp.concatenate(keeps, axis=0)
        lt_incl = jnp.where(sub128 <= lane128, 1.0, 0.0)
        cs = jnp.dot(keep_mat, lt_incl, preferred_element_type=jnp.float32)
        rowsum = jnp.dot(keep_mat, jnp.ones((128, 1), jnp.float32),
                         preferred_element_type=jnp.float32)
        subB = jax.lax.broadcasted_iota(jnp.int32, (NBLK, 1), 0)
        laneB = jax.lax.broadcasted_iota(jnp.int32, (1, NBLK), 1)
        strictB = jnp.where(laneB < subB, 1.0, 0.0)
        offs = jnp.dot(strictB, rowsum, preferred_element_type=jnp.float32)
        pack_incl = cs + offs
        pt = jnp.where(keep_mat > 0.0, pack_incl - 1.0, float(DUMP)).astype(jnp.int32)
        pt_o[...] = pt[None]
        nk_o[...] = jnp.sum(keep_mat).reshape(1, 1)[None]

    return pl.pallas_call(
        body,
        grid=(B,),
        in_specs=[
            pl.BlockSpec((1, NBLK, 128), lambda b: (b, 0, 0)),
            pl.BlockSpec((1, 128, NBLK), lambda b: (b, 0, 0)),
            pl.BlockSpec((1, N, 5), lambda b: (b, 0, 0)),
            pl.BlockSpec((1, 5, N), lambda b: (b, 0, 0)),
        ],
        out_specs=(
            pl.BlockSpec((1, NBLK, 128), lambda b: (b, 0, 0)),
            pl.BlockSpec((1, 1, 1), lambda b: (b, 0, 0)),
        ),
        out_shape=(
            jax.ShapeDtypeStruct((B, NBLK, 128), jnp.int32),
            jax.ShapeDtypeStruct((B, 1, 1), jnp.float32),
        ),
    )(srt_mat, srt_matT, rois, roisT)


def _sc_pack_scatter(pt, sg, ss):
    mesh = plsc.VectorSubcoreMesh(core_axis_name="c", subcore_axis_name="s")

    @functools.partial(
        pl.kernel, mesh=mesh,
        compiler_params=pltpu.CompilerParams(needs_layout_passes=False),
        out_type=(jax.ShapeDtypeStruct((B, KP + 128), jnp.int32),
                  jax.ShapeDtypeStruct((B, KP + 128), jnp.float32)),
        scratch_types=[pltpu.VMEM((KP,), jnp.int32),
                       pltpu.VMEM((KP,), jnp.int32),
                       pltpu.VMEM((KP,), jnp.float32),
                       pltpu.VMEM((KP + 128,), jnp.int32),
                       pltpu.VMEM((KP + 128,), jnp.float32)],
    )
    def k(pt_hbm, sg_hbm, ss_hbm, og_hbm, os_hbm, pt_v, sg_v, ss_v, og_v, os_v):
        wid = lax.axis_index("s") * 2 + lax.axis_index("c")

        @pl.when(wid < B)
        def _():
            pltpu.sync_copy(pt_hbm.at[wid], pt_v)
            pltpu.sync_copy(sg_hbm.at[wid], sg_v)
            pltpu.sync_copy(ss_hbm.at[wid], ss_v)

            def zbody(c, carry):
                z16 = jnp.zeros((16,), jnp.int32)
                og_v[pl.ds(c * 16, 16)] = z16
                os_v[pl.ds(c * 16, 16)] = z16.astype(jnp.float32)
                return carry

            lax.fori_loop(0, (KP + 128) // 16, zbody, 0)

            def body(c, carry):
                idx = pt_v[pl.ds(c * 16, 16)]
                msk = idx < KP
                idxc = jnp.minimum(idx, KP + 15)
                gv = sg_v[pl.ds(c * 16, 16)]
                plsc.store_scatter(og_v, [idxc], gv, mask=msk)
                sv = ss_v[pl.ds(c * 16, 16)]
                plsc.store_scatter(os_v, [idxc], sv, mask=msk)
                return carry

            lax.fori_loop(0, KP // 16, body, 0)
            pltpu.sync_copy(og_v, og_hbm.at[wid])
            pltpu.sync_copy(os_v, os_hbm.at[wid])

    return k(pt, sg, ss)


def _assembly_call(pg3, ps3, nk, rois):
    P = N * N - N
    M = KP + 128

    TA = 1024
    NCH = 10

    def body(pg_r, ps_r, nk_r, rois_r, o9_r, opr_r, osc_r):
        b = pl.program_id(0)
        c = pl.program_id(1)
        pg_col = pg_r[0]
        psc = ps_r[0]
        nkept = nk_r[0]
        r5 = rois_r[0]
        pos = c * TA + jax.lax.broadcasted_iota(jnp.int32, (TA, 1), 0)
        vflat = jnp.where(pos.astype(jnp.float32) < nkept[0, 0], 1.0, 0.0)
        vi = vflat > 0.0
        pi_flat = pg_col // N
        pj_flat = pg_col % N
        iota_rowN = jax.lax.broadcasted_iota(jnp.int32, (1, N), 1)
        OHi = jnp.where(pi_flat == iota_rowN, 1.0, 0.0)
        OHj = jnp.where(pj_flat == iota_rowN, 1.0, 0.0)
        boxes = r5[:, 1:5]
        bi = jnp.dot(OHi, boxes, preferred_element_type=jnp.float32,
                     precision=jax.lax.Precision.HIGHEST)
        bj = jnp.dot(OHj, boxes, preferred_element_type=jnp.float32,
                     precision=jax.lax.Precision.HIGHEST)
        col0 = jnp.where(vi, jnp.float32(b), 0.0)
        o9_r[...] = jnp.concatenate([col0, bi * vflat, bj * vflat], axis=1)[None]
        opr_r[...] = jnp.concatenate(
            [jnp.where(vi, pi_flat, 0), jnp.where(vi, pj_flat, 0)], axis=1)[None]
        osc_r[...] = jnp.where(vi, psc, 0.0)[None]

    return pl.pallas_call(
        body,
        grid=(B, NCH),
        in_specs=[
            pl.BlockSpec((1, TA, 1), lambda b, c: (b, jnp.minimum(c, (KP + 128) // TA - 1), 0)),
            pl.BlockSpec((1, TA, 1), lambda b, c: (b, jnp.minimum(c, (KP + 128) // TA - 1), 0)),
            pl.BlockSpec((1, 1, 1), lambda b, c: (b, 0, 0)),
            pl.BlockSpec((1, N, 5), lambda b, c: (b, 0, 0)),
        ],
        out_specs=(
            pl.BlockSpec((1, TA, 9), lambda b, c: (b, c, 0)),
            pl.BlockSpec((1, TA, 2), lambda b, c: (b, c, 0)),
            pl.BlockSpec((1, TA, 1), lambda b, c: (b, c, 0)),
        ),
        out_shape=(
            jax.ShapeDtypeStruct((B, NCH * TA, 9), jnp.float32),
            jax.ShapeDtypeStruct((B, NCH * TA, 2), jnp.int32),
            jax.ShapeDtypeStruct((B, NCH * TA, 1), jnp.float32),
        ),
    )(pg3, ps3, nk, rois)


def kernel(rois, im_info, roi_feat, nlp_features, W_sub, b_sub, W_ih_f, W_hh_f, b_ih_f, b_hh_f, W_ih_b, W_hh_b, b_ih_b, b_hh_b, W_lo, b_lo):
    scores_all = _scores_pallas(roi_feat, nlp_features, W_sub, b_sub, W_ih_f, W_hh_f,
                                b_ih_f, b_hh_f, W_ih_b, W_hh_b, b_ih_b, b_hh_b, W_lo, b_lo)
    scores_pad = jnp.concatenate(
        [scores_all, jnp.full((B, RPAD - N * N), NEG, jnp.float32)], axis=1)
    rank = _rank_call(scores_pad.reshape(B, RPAD, 1), scores_pad.reshape(B, 1, RPAD))
    srt_g, srt_s = _sc_rank_scatter(rank.reshape(B, RPAD), scores_pad)
    sg6 = srt_g[:, :KP]
    ss6 = srt_s[:, :KP]
    srt_mat = sg6.reshape(B, NBLK, 128)
    srt_matT = jnp.transpose(srt_mat, (0, 2, 1))
    pt, nk = _nms_call(srt_mat, srt_matT, rois, jnp.transpose(rois, (0, 2, 1)))
    pg, ps = _sc_pack_scatter(pt.reshape(B, KP), sg6, ss6)
    o9p, oprp, oscp = _assembly_call(pg.reshape(B, KP + 128, 1), ps.reshape(B, KP + 128, 1),
                                     nk, rois)
    P = N * N - N
    return o9p[:, :P, :], oprp[:, :P, :], oscp[:, :P, :]
